# Initial kernel scaffold; baseline (speedup 1.0000x reference)
import functools

import jax
import jax.numpy as jnp
from jax import lax
from jax.experimental import pallas as pl
from jax.experimental.pallas import tpu as pltpu
from jax.experimental.pallas import tpu_sc as plsc

N = 2048
D = 1024
E = 8
K = 2
EMB = 1024
H1 = 2048
H2 = 1024
DO = 1024

RBLK = 256
BLK = 128
MAXPAD = 5120
NBLK = MAXPAD // BLK

NC, NS = 2, 16
NW = NC * NS


def _router_body(x_ref, wr_ref, br_ref, embt_ref, i0_ref, i1_ref, g0_ref,
                 g1_ref):
    x = x_ref[...]
    q = jnp.dot(x, wr_ref[...], preferred_element_type=jnp.float32)
    q = q + br_ref[...]
    dots = jnp.dot(q, embt_ref[...], preferred_element_type=jnp.float32)
    qn = jnp.sum(q * q, axis=1, keepdims=True)
    en = jnp.sum(embt_ref[...] * embt_ref[...], axis=0, keepdims=True)
    scores = 2.0 * dots - qn - en
    iota = lax.broadcasted_iota(jnp.int32, scores.shape, 1)
    neginf = jnp.float32(-jnp.inf)
    scores = jnp.where(iota < E, scores, neginf)
    m1 = jnp.max(scores, axis=1, keepdims=True)
    a1 = jnp.min(jnp.where(scores == m1, iota, E), axis=1, keepdims=True)
    masked = jnp.where(iota == a1, neginf, scores)
    m2 = jnp.max(masked, axis=1, keepdims=True)
    a2 = jnp.min(jnp.where(masked == m2, iota, E), axis=1, keepdims=True)
    g = 1.0 / (1.0 + jnp.exp(m2 - m1))
    i0_ref[...] = a1
    i1_ref[...] = a2
    g0_ref[...] = g
    g1_ref[...] = 1.0 - g


def _run_router(x, wr, br, emb):
    embt = jnp.zeros((EMB, 128), jnp.float32).at[:, :E].set(emb.T)
    out_shapes = (
        jax.ShapeDtypeStruct((N, 1), jnp.int32),
        jax.ShapeDtypeStruct((N, 1), jnp.int32),
        jax.ShapeDtypeStruct((N, 1), jnp.float32),
        jax.ShapeDtypeStruct((N, 1), jnp.float32),
    )
    ospec = pl.BlockSpec((RBLK, 1), lambda i: (i, 0))
    return pl.pallas_call(
        _router_body,
        grid=(N // RBLK,),
        in_specs=[
            pl.BlockSpec((RBLK, D), lambda i: (i, 0)),
            pl.BlockSpec((D, EMB), lambda i: (0, 0)),
            pl.BlockSpec((1, EMB), lambda i: (0, 0)),
            pl.BlockSpec((EMB, 128), lambda i: (0, 0)),
        ],
        out_specs=(ospec, ospec, ospec, ospec),
        out_shape=out_shapes,
    )(x, wr, br.reshape(1, EMB), embt)


def _route_metadata(i0, i1, g0, g1):
    flat_e = jnp.concatenate([i0, i1], axis=1).reshape(N * K)
    flat_g = jnp.concatenate([g0, g1], axis=1).reshape(N * K)
    oh = (flat_e[:, None] == jnp.arange(E)[None, :]).astype(jnp.int32)
    cum = jnp.cumsum(oh, axis=0)
    counts = cum[-1]
    rank = jnp.take_along_axis(cum, flat_e[:, None], axis=1).reshape(-1) - 1
    padded = ((counts + BLK - 1) // BLK) * BLK
    cum_pad = jnp.cumsum(padded)
    pad_start = cum_pad - padded
    padded_pos = (pad_start[flat_e] + rank).astype(jnp.int32)
    tok = jnp.arange(N * K, dtype=jnp.int32) // K
    row_token = jnp.zeros((MAXPAD,), jnp.int32).at[padded_pos].set(tok)
    row_gate = jnp.zeros((MAXPAD,), jnp.float32).at[padded_pos].set(flat_g)
    blk_e = jnp.searchsorted(cum_pad, jnp.arange(NBLK) * BLK, side="right")
    blk_e = jnp.minimum(blk_e, E - 1).astype(jnp.int32)
    pp = padded_pos.reshape(N, K)
    return row_token, row_gate, blk_e, pp[:, 0], pp[:, 1]


def _sc_gather(x, row_token):
    rows_per_w = MAXPAD // NW
    ch = rows_per_w // 2
    mesh = plsc.VectorSubcoreMesh(core_axis_name="c", subcore_axis_name="s",
                                  num_cores=NC, num_subcores=NS)

    @functools.partial(
        pl.kernel, mesh=mesh,
        out_type=jax.ShapeDtypeStruct((MAXPAD, D), jnp.float32),
        scratch_types=[
            pltpu.VMEM((ch,), jnp.int32),
            pltpu.VMEM((ch, D), jnp.float32),
            pltpu.SemaphoreType.DMA,
        ],
    )
    def k(x_hbm, idx_hbm, out_hbm, idx_v, rows_v, sem):
        wid = lax.axis_index("s") * NC + lax.axis_index("c")
        for c in range(2):
            base = wid * rows_per_w + c * ch
            pltpu.sync_copy(idx_hbm.at[pl.ds(base, ch)], idx_v)
            pltpu.async_copy(x_hbm.at[idx_v], rows_v, sem).wait()
            pltpu.sync_copy(rows_v, out_hbm.at[pl.ds(base, ch)])

    return k(x, row_token)


def _mlp_body(be_ref, xs_ref, gate_ref, w1_ref, b1_ref, w2_ref, b2_ref,
              w3_ref, b3_ref, out_ref):
    x = xs_ref[...]
    h = jnp.dot(x, w1_ref[0], preferred_element_type=jnp.float32)
    h = jnp.maximum(h + b1_ref[0], 0.0)
    h = jnp.dot(h, w2_ref[0], preferred_element_type=jnp.float32)
    h = jnp.maximum(h + b2_ref[0], 0.0)
    y = jnp.dot(h, w3_ref[0], preferred_element_type=jnp.float32)
    y = y + b3_ref[0]
    out_ref[...] = y * gate_ref[...]


def _run_mlp(xs, row_gate, blk_e, w1, b1, w2, b2, w3, b3):
    grid_spec = pltpu.PrefetchScalarGridSpec(
        num_scalar_prefetch=1,
        grid=(NBLK,),
        in_specs=[
            pl.BlockSpec((BLK, D), lambda i, be: (i, 0)),
            pl.BlockSpec((BLK, 1), lambda i, be: (i, 0)),
            pl.BlockSpec((1, D, H1), lambda i, be: (be[i], 0, 0)),
            pl.BlockSpec((1, 1, H1), lambda i, be: (be[i], 0, 0)),
            pl.BlockSpec((1, H1, H2), lambda i, be: (be[i], 0, 0)),
            pl.BlockSpec((1, 1, H2), lambda i, be: (be[i], 0, 0)),
            pl.BlockSpec((1, H2, DO), lambda i, be: (be[i], 0, 0)),
            pl.BlockSpec((1, 1, DO), lambda i, be: (be[i], 0, 0)),
        ],
        out_specs=pl.BlockSpec((BLK, DO), lambda i, be: (i, 0)),
    )
    return pl.pallas_call(
        _mlp_body,
        grid_spec=grid_spec,
        out_shape=jax.ShapeDtypeStruct((MAXPAD, DO), jnp.float32),
    )(blk_e, xs, row_gate.reshape(MAXPAD, 1),
      w1, b1.reshape(E, 1, H1), w2, b2.reshape(E, 1, H2),
      w3, b3.reshape(E, 1, DO))


def _sc_combine(ys, pos0, pos1):
    tok_per_w = N // NW
    ch = tok_per_w // 2
    mesh = plsc.VectorSubcoreMesh(core_axis_name="c", subcore_axis_name="s",
                                  num_cores=NC, num_subcores=NS)

    @functools.partial(
        pl.kernel, mesh=mesh,
        out_type=jax.ShapeDtypeStruct((N, DO), jnp.float32),
        scratch_types=[
            pltpu.VMEM((ch,), jnp.int32),
            pltpu.VMEM((ch,), jnp.int32),
            pltpu.VMEM((ch, DO), jnp.float32),
            pltpu.VMEM((ch, DO), jnp.float32),
            pltpu.SemaphoreType.DMA,
        ],
    )
    def k(ys_hbm, p0_hbm, p1_hbm, out_hbm, i0_v, i1_v, r0_v, r1_v, sem):
        wid = lax.axis_index("s") * NC + lax.axis_index("c")
        for c in range(2):
            base = wid * tok_per_w + c * ch
            pltpu.sync_copy(p0_hbm.at[pl.ds(base, ch)], i0_v)
            pltpu.sync_copy(p1_hbm.at[pl.ds(base, ch)], i1_v)
            pltpu.async_copy(ys_hbm.at[i0_v], r0_v, sem).wait()
            pltpu.async_copy(ys_hbm.at[i1_v], r1_v, sem).wait()

            def row_body(r, _):
                def col_body(cc, _):
                    s = pl.ds(cc * 16, 16)
                    r0_v[r, s] = r0_v[r, s] + r1_v[r, s]
                    return 0
                return lax.fori_loop(0, DO // 16, col_body, 0)

            lax.fori_loop(0, ch, row_body, 0)
            pltpu.sync_copy(r0_v, out_hbm.at[pl.ds(base, ch)])

    return k(ys, pos0, pos1)


def kernel(inputs, Wr, br, expert_embeddings, W1, b1, W2, b2, W3, b3):
    i0, i1, g0, g1 = _run_router(inputs, Wr, br, expert_embeddings)
    row_token, row_gate, blk_e, pos0, pos1 = _route_metadata(i0, i1, g0, g1)
    xs = _sc_gather(inputs, row_token)
    ys = _run_mlp(xs, row_gate, blk_e, W1, b1, W2, b2, W3, b3)
    return _sc_combine(ys, pos0, pos1)


# speedup vs baseline: 1.4628x; 1.4628x over previous
import functools

import jax
import jax.numpy as jnp
from jax import lax
from jax.experimental import pallas as pl
from jax.experimental.pallas import tpu as pltpu
from jax.experimental.pallas import tpu_sc as plsc

N = 2048
D = 1024
E = 8
K = 2
EMB = 1024
H1 = 2048
H2 = 1024
DO = 1024

RBLK = 256
BLK = 128
MAXPAD = 5120
NBLK = MAXPAD // BLK

NC, NS = 2, 16
NW = NC * NS


def _router_body(x_ref, wr_ref, br_ref, emb_ref, i0_ref, i1_ref, g0_ref,
                 g1_ref):
    x = x_ref[...]
    q = jnp.dot(x.astype(jnp.bfloat16), wr_ref[...].astype(jnp.bfloat16),
                preferred_element_type=jnp.float32)
    q = q + br_ref[...]
    cols = []
    for e in range(E):
        de = q - emb_ref[e, :][None, :]
        cols.append(-jnp.sum(de * de, axis=1, keepdims=True))
    scores = jnp.concatenate(cols, axis=1)
    iota = lax.broadcasted_iota(jnp.int32, scores.shape, 1)
    neginf = jnp.float32(-jnp.inf)
    m1 = jnp.max(scores, axis=1, keepdims=True)
    a1 = jnp.min(jnp.where(scores == m1, iota, E), axis=1, keepdims=True)
    masked = jnp.where(iota == a1, neginf, scores)
    m2 = jnp.max(masked, axis=1, keepdims=True)
    a2 = jnp.min(jnp.where(masked == m2, iota, E), axis=1, keepdims=True)
    g = 1.0 / (1.0 + jnp.exp(m2 - m1))
    i0_ref[...] = a1
    i1_ref[...] = a2
    g0_ref[...] = g
    g1_ref[...] = 1.0 - g


def _run_router(x, wr, br, emb):
    out_shapes = (
        jax.ShapeDtypeStruct((N, 1), jnp.int32),
        jax.ShapeDtypeStruct((N, 1), jnp.int32),
        jax.ShapeDtypeStruct((N, 1), jnp.float32),
        jax.ShapeDtypeStruct((N, 1), jnp.float32),
    )
    ospec = pl.BlockSpec((RBLK, 1), lambda i: (i, 0))
    return pl.pallas_call(
        _router_body,
        grid=(N // RBLK,),
        in_specs=[
            pl.BlockSpec((RBLK, D), lambda i: (i, 0)),
            pl.BlockSpec((D, EMB), lambda i: (0, 0)),
            pl.BlockSpec((1, EMB), lambda i: (0, 0)),
            pl.BlockSpec((E, EMB), lambda i: (0, 0)),
        ],
        out_specs=(ospec, ospec, ospec, ospec),
        out_shape=out_shapes,
    )(x, wr, br.reshape(1, EMB), emb)


def _route_metadata(i0, i1, g0, g1):
    flat_e = jnp.concatenate([i0, i1], axis=1).reshape(N * K)
    flat_g = jnp.concatenate([g0, g1], axis=1).reshape(N * K)
    oh = (flat_e[:, None] == jnp.arange(E)[None, :]).astype(jnp.int32)
    cum = jnp.cumsum(oh, axis=0)
    counts = cum[-1]
    rank = jnp.take_along_axis(cum, flat_e[:, None], axis=1).reshape(-1) - 1
    padded = ((counts + BLK - 1) // BLK) * BLK
    cum_pad = jnp.cumsum(padded)
    pad_start = cum_pad - padded
    padded_pos = (pad_start[flat_e] + rank).astype(jnp.int32)
    tok = jnp.arange(N * K, dtype=jnp.int32) // K
    row_token = jnp.zeros((MAXPAD,), jnp.int32).at[padded_pos].set(tok)
    row_gate = jnp.zeros((MAXPAD,), jnp.float32).at[padded_pos].set(flat_g)
    blk_e = jnp.searchsorted(cum_pad, jnp.arange(NBLK) * BLK, side="right")
    blk_e = jnp.minimum(blk_e, E - 1).astype(jnp.int32)
    pp = padded_pos.reshape(N, K)
    return row_token, row_gate, blk_e, pp[:, 0], pp[:, 1]


def _sc_gather(x, row_token):
    rows_per_w = MAXPAD // NW
    ch = rows_per_w // 2
    mesh = plsc.VectorSubcoreMesh(core_axis_name="c", subcore_axis_name="s",
                                  num_cores=NC, num_subcores=NS)

    @functools.partial(
        pl.kernel, mesh=mesh,
        out_type=jax.ShapeDtypeStruct((MAXPAD, D), jnp.float32),
        scratch_types=[
            pltpu.VMEM((ch,), jnp.int32),
            pltpu.VMEM((ch, D), jnp.float32),
            pltpu.SemaphoreType.DMA,
        ],
    )
    def k(x_hbm, idx_hbm, out_hbm, idx_v, rows_v, sem):
        wid = lax.axis_index("s") * NC + lax.axis_index("c")
        for c in range(2):
            base = wid * rows_per_w + c * ch
            pltpu.sync_copy(idx_hbm.at[pl.ds(base, ch)], idx_v)
            pltpu.async_copy(x_hbm.at[idx_v], rows_v, sem).wait()
            pltpu.sync_copy(rows_v, out_hbm.at[pl.ds(base, ch)])

    return k(x, row_token)


def _mlp_body(be_ref, xs_ref, gate_ref, w1_ref, b1_ref, w2_ref, b2_ref,
              w3_ref, b3_ref, out_ref):
    bf = jnp.bfloat16
    x = xs_ref[...]
    h = jnp.dot(x.astype(bf), w1_ref[0].astype(bf),
                preferred_element_type=jnp.float32)
    h = jnp.maximum(h + b1_ref[0], 0.0)
    h = jnp.dot(h.astype(bf), w2_ref[0].astype(bf),
                preferred_element_type=jnp.float32)
    h = jnp.maximum(h + b2_ref[0], 0.0)
    y = jnp.dot(h.astype(bf), w3_ref[0].astype(bf),
                preferred_element_type=jnp.float32)
    y = y + b3_ref[0]
    out_ref[...] = y * gate_ref[...]


def _run_mlp(xs, row_gate, blk_e, w1, b1, w2, b2, w3, b3):
    grid_spec = pltpu.PrefetchScalarGridSpec(
        num_scalar_prefetch=1,
        grid=(NBLK,),
        in_specs=[
            pl.BlockSpec((BLK, D), lambda i, be: (i, 0)),
            pl.BlockSpec((BLK, 1), lambda i, be: (i, 0)),
            pl.BlockSpec((1, D, H1), lambda i, be: (be[i], 0, 0)),
            pl.BlockSpec((1, 1, H1), lambda i, be: (be[i], 0, 0)),
            pl.BlockSpec((1, H1, H2), lambda i, be: (be[i], 0, 0)),
            pl.BlockSpec((1, 1, H2), lambda i, be: (be[i], 0, 0)),
            pl.BlockSpec((1, H2, DO), lambda i, be: (be[i], 0, 0)),
            pl.BlockSpec((1, 1, DO), lambda i, be: (be[i], 0, 0)),
        ],
        out_specs=pl.BlockSpec((BLK, DO), lambda i, be: (i, 0)),
    )
    return pl.pallas_call(
        _mlp_body,
        grid_spec=grid_spec,
        out_shape=jax.ShapeDtypeStruct((MAXPAD, DO), jnp.float32),
    )(blk_e, xs, row_gate.reshape(MAXPAD, 1),
      w1, b1.reshape(E, 1, H1), w2, b2.reshape(E, 1, H2),
      w3, b3.reshape(E, 1, DO))


def _sc_combine(ys, pos0, pos1):
    tok_per_w = N // NW
    ch = tok_per_w // 2
    mesh = plsc.VectorSubcoreMesh(core_axis_name="c", subcore_axis_name="s",
                                  num_cores=NC, num_subcores=NS)

    @functools.partial(
        pl.kernel, mesh=mesh,
        out_type=jax.ShapeDtypeStruct((N, DO), jnp.float32),
        scratch_types=[
            pltpu.VMEM((ch,), jnp.int32),
            pltpu.VMEM((ch,), jnp.int32),
            pltpu.VMEM((ch, DO), jnp.float32),
            pltpu.VMEM((ch, DO), jnp.float32),
            pltpu.SemaphoreType.DMA,
        ],
    )
    def k(ys_hbm, p0_hbm, p1_hbm, out_hbm, i0_v, i1_v, r0_v, r1_v, sem):
        wid = lax.axis_index("s") * NC + lax.axis_index("c")
        for c in range(2):
            base = wid * tok_per_w + c * ch
            pltpu.sync_copy(p0_hbm.at[pl.ds(base, ch)], i0_v)
            pltpu.sync_copy(p1_hbm.at[pl.ds(base, ch)], i1_v)
            pltpu.async_copy(ys_hbm.at[i0_v], r0_v, sem).wait()
            pltpu.async_copy(ys_hbm.at[i1_v], r1_v, sem).wait()

            def row_body(r, _):
                def col_body(cc, _):
                    s = pl.ds(cc * 16, 16)
                    r0_v[r, s] = r0_v[r, s] + r1_v[r, s]
                    return 0
                return lax.fori_loop(0, DO // 16, col_body, 0)

            lax.fori_loop(0, ch, row_body, 0)
            pltpu.sync_copy(r0_v, out_hbm.at[pl.ds(base, ch)])

    return k(ys, pos0, pos1)


def kernel(inputs, Wr, br, expert_embeddings, W1, b1, W2, b2, W3, b3):
    i0, i1, g0, g1 = _run_router(inputs, Wr, br, expert_embeddings)
    row_token, row_gate, blk_e, pos0, pos1 = _route_metadata(i0, i1, g0, g1)
    xs = _sc_gather(inputs, row_token)
    ys = _run_mlp(xs, row_gate, blk_e, W1, b1, W2, b2, W3, b3)
    return _sc_combine(ys, pos0, pos1)


# speedup vs baseline: 1.5494x; 1.0592x over previous
import functools

import jax
import jax.numpy as jnp
from jax import lax
from jax.experimental import pallas as pl
from jax.experimental.pallas import tpu as pltpu
from jax.experimental.pallas import tpu_sc as plsc

N = 2048
D = 1024
E = 8
K = 2
EMB = 1024
H1 = 2048
H2 = 1024
DO = 1024

RBLK = 256
BLK = 128
MAXPAD = 5120
NBLK = MAXPAD // BLK

NC, NS = 2, 16
NW = NC * NS


def _router_body(x_ref, wr_ref, br_ref, emb_ref, i0_ref, i1_ref, g0_ref,
                 g1_ref):
    x = x_ref[...]
    q = jnp.dot(x.astype(jnp.bfloat16), wr_ref[...].astype(jnp.bfloat16),
                preferred_element_type=jnp.float32)
    q = q + br_ref[...]
    cols = []
    for e in range(E):
        de = q - emb_ref[e, :][None, :]
        cols.append(-jnp.sum(de * de, axis=1, keepdims=True))
    scores = jnp.concatenate(cols, axis=1)
    iota = lax.broadcasted_iota(jnp.int32, scores.shape, 1)
    neginf = jnp.float32(-jnp.inf)
    m1 = jnp.max(scores, axis=1, keepdims=True)
    a1 = jnp.min(jnp.where(scores == m1, iota, E), axis=1, keepdims=True)
    masked = jnp.where(iota == a1, neginf, scores)
    m2 = jnp.max(masked, axis=1, keepdims=True)
    a2 = jnp.min(jnp.where(masked == m2, iota, E), axis=1, keepdims=True)
    g = 1.0 / (1.0 + jnp.exp(m2 - m1))
    i0_ref[...] = a1
    i1_ref[...] = a2
    g0_ref[...] = g
    g1_ref[...] = 1.0 - g


def _run_router(x, wr, br, emb):
    out_shapes = (
        jax.ShapeDtypeStruct((N, 1), jnp.int32),
        jax.ShapeDtypeStruct((N, 1), jnp.int32),
        jax.ShapeDtypeStruct((N, 1), jnp.float32),
        jax.ShapeDtypeStruct((N, 1), jnp.float32),
    )
    ospec = pl.BlockSpec((RBLK, 1), lambda i: (i, 0))
    return pl.pallas_call(
        _router_body,
        grid=(N // RBLK,),
        in_specs=[
            pl.BlockSpec((RBLK, D), lambda i: (i, 0)),
            pl.BlockSpec((D, EMB), lambda i: (0, 0)),
            pl.BlockSpec((1, EMB), lambda i: (0, 0)),
            pl.BlockSpec((E, EMB), lambda i: (0, 0)),
        ],
        out_specs=(ospec, ospec, ospec, ospec),
        out_shape=out_shapes,
    )(x, wr, br.reshape(1, EMB), emb)


def _route_metadata(i0, i1, g0, g1):
    flat_e = jnp.concatenate([i0, i1], axis=1).reshape(N * K)
    flat_g = jnp.concatenate([g0, g1], axis=1).reshape(N * K)
    oh = (flat_e[:, None] == jnp.arange(E)[None, :]).astype(jnp.int32)
    cum = jnp.cumsum(oh, axis=0)
    counts = cum[-1]
    rank = jnp.take_along_axis(cum, flat_e[:, None], axis=1).reshape(-1) - 1
    padded = ((counts + BLK - 1) // BLK) * BLK
    cum_pad = jnp.cumsum(padded)
    pad_start = cum_pad - padded
    padded_pos = (pad_start[flat_e] + rank).astype(jnp.int32)
    tok = jnp.arange(N * K, dtype=jnp.int32) // K
    row_token = jnp.zeros((MAXPAD,), jnp.int32).at[padded_pos].set(tok)
    row_gate = jnp.zeros((MAXPAD,), jnp.float32).at[padded_pos].set(flat_g)
    blk_e = jnp.searchsorted(cum_pad, jnp.arange(NBLK) * BLK, side="right")
    blk_e = jnp.minimum(blk_e, E - 1).astype(jnp.int32)
    pp = padded_pos.reshape(N, K)
    return row_token, row_gate, blk_e, pp[:, 0], pp[:, 1]


def _sc_gather(x, row_token):
    rows_per_w = MAXPAD // NW
    ch = 40
    nch = rows_per_w // ch
    mesh = plsc.VectorSubcoreMesh(core_axis_name="c", subcore_axis_name="s",
                                  num_cores=NC, num_subcores=NS)

    @functools.partial(
        pl.kernel, mesh=mesh,
        out_type=jax.ShapeDtypeStruct((MAXPAD, D), jnp.float32),
        scratch_types=[
            pltpu.VMEM((rows_per_w,), jnp.int32),
            pltpu.VMEM((ch, D), jnp.float32),
            pltpu.VMEM((ch, D), jnp.float32),
            pltpu.SemaphoreType.DMA,
            pltpu.SemaphoreType.DMA,
            pltpu.SemaphoreType.DMA,
            pltpu.SemaphoreType.DMA,
        ],
    )
    def k(x_hbm, idx_hbm, out_hbm, idx_v, buf0, buf1, g0, g1, s0, s1):
        wid = lax.axis_index("s") * NC + lax.axis_index("c")
        base = wid * rows_per_w
        pltpu.sync_copy(idx_hbm.at[pl.ds(base, rows_per_w)], idx_v)
        bufs, gsems, ssems = (buf0, buf1), (g0, g1), (s0, s1)
        gd = [pltpu.async_copy(x_hbm.at[idx_v.at[pl.ds(c * ch, ch)]],
                               bufs[c], gsems[c]) for c in range(2)]
        stores = [None, None]
        for c in range(nch):
            b = c & 1
            gd[c].wait()
            sd = pltpu.async_copy(bufs[b], out_hbm.at[pl.ds(base + c * ch, ch)],
                                  ssems[b])
            stores[b] = sd
            if c + 2 < nch:
                sd.wait()
                gd.append(pltpu.async_copy(
                    x_hbm.at[idx_v.at[pl.ds((c + 2) * ch, ch)]],
                    bufs[b], gsems[b]))
        stores[0].wait()
        stores[1].wait()

    return k(x, row_token)


def _mlp_body(be_ref, xs_ref, gate_ref, w1_ref, b1_ref, w2_ref, b2_ref,
              w3_ref, b3_ref, out_ref):
    bf = jnp.bfloat16
    x = xs_ref[...]
    h = jnp.dot(x.astype(bf), w1_ref[0].astype(bf),
                preferred_element_type=jnp.float32)
    h = jnp.maximum(h + b1_ref[0], 0.0)
    h = jnp.dot(h.astype(bf), w2_ref[0].astype(bf),
                preferred_element_type=jnp.float32)
    h = jnp.maximum(h + b2_ref[0], 0.0)
    y = jnp.dot(h.astype(bf), w3_ref[0].astype(bf),
                preferred_element_type=jnp.float32)
    y = y + b3_ref[0]
    out_ref[...] = y * gate_ref[...]


def _run_mlp(xs, row_gate, blk_e, w1, b1, w2, b2, w3, b3):
    grid_spec = pltpu.PrefetchScalarGridSpec(
        num_scalar_prefetch=1,
        grid=(NBLK,),
        in_specs=[
            pl.BlockSpec((BLK, D), lambda i, be: (i, 0)),
            pl.BlockSpec((BLK, 1), lambda i, be: (i, 0)),
            pl.BlockSpec((1, D, H1), lambda i, be: (be[i], 0, 0)),
            pl.BlockSpec((1, 1, H1), lambda i, be: (be[i], 0, 0)),
            pl.BlockSpec((1, H1, H2), lambda i, be: (be[i], 0, 0)),
            pl.BlockSpec((1, 1, H2), lambda i, be: (be[i], 0, 0)),
            pl.BlockSpec((1, H2, DO), lambda i, be: (be[i], 0, 0)),
            pl.BlockSpec((1, 1, DO), lambda i, be: (be[i], 0, 0)),
        ],
        out_specs=pl.BlockSpec((BLK, DO), lambda i, be: (i, 0)),
    )
    return pl.pallas_call(
        _mlp_body,
        grid_spec=grid_spec,
        out_shape=jax.ShapeDtypeStruct((MAXPAD, DO), jnp.float32),
    )(blk_e, xs, row_gate.reshape(MAXPAD, 1),
      w1, b1.reshape(E, 1, H1), w2, b2.reshape(E, 1, H2),
      w3, b3.reshape(E, 1, DO))


def _sc_combine(ys, pos0, pos1):
    tok_per_w = N // NW
    ch = 16
    mesh = plsc.VectorSubcoreMesh(core_axis_name="c", subcore_axis_name="s",
                                  num_cores=NC, num_subcores=NS)

    @functools.partial(
        pl.kernel, mesh=mesh,
        out_type=jax.ShapeDtypeStruct((N, DO), jnp.float32),
        scratch_types=[
            pltpu.VMEM((tok_per_w,), jnp.int32),
            pltpu.VMEM((tok_per_w,), jnp.int32),
            pltpu.VMEM((ch, DO), jnp.float32),
            pltpu.VMEM((ch, DO), jnp.float32),
            pltpu.VMEM((ch, DO), jnp.float32),
            pltpu.VMEM((ch, DO), jnp.float32),
            pltpu.SemaphoreType.DMA,
            pltpu.SemaphoreType.DMA,
            pltpu.SemaphoreType.DMA,
            pltpu.SemaphoreType.DMA,
            pltpu.SemaphoreType.DMA,
            pltpu.SemaphoreType.DMA,
        ],
    )
    def k(ys_hbm, p0_hbm, p1_hbm, out_hbm, i0_v, i1_v, a0, a1, b0, b1,
          ga0, ga1, gb0, gb1, s0, s1):
        wid = lax.axis_index("s") * NC + lax.axis_index("c")
        base = wid * tok_per_w
        pltpu.sync_copy(p0_hbm.at[pl.ds(base, tok_per_w)], i0_v)
        pltpu.sync_copy(p1_hbm.at[pl.ds(base, tok_per_w)], i1_v)
        abufs, bbufs = (a0, a1), (b0, b1)
        gasems, gbsems, ssems = (ga0, ga1), (gb0, gb1), (s0, s1)
        nch = tok_per_w // ch
        d0 = pltpu.async_copy(ys_hbm.at[i0_v.at[pl.ds(0, ch)]], a0, ga0)
        d1 = pltpu.async_copy(ys_hbm.at[i1_v.at[pl.ds(0, ch)]], b0, gb0)
        stores = [None, None]
        for c in range(nch):
            b = c & 1
            nd0 = nd1 = None
            if c + 1 < nch:
                ob = (c + 1) & 1
                if stores[ob] is not None:
                    stores[ob].wait()
                sl = pl.ds((c + 1) * ch, ch)
                nd0 = pltpu.async_copy(ys_hbm.at[i0_v.at[sl]], abufs[ob],
                                       gasems[ob])
                nd1 = pltpu.async_copy(ys_hbm.at[i1_v.at[sl]], bbufs[ob],
                                       gbsems[ob])
            d0.wait()
            d1.wait()
            a_ref, b_ref = abufs[b], bbufs[b]

            def row_body(r, _, a_ref=a_ref, b_ref=b_ref):
                for cc in range(DO // 16):
                    s = pl.ds(cc * 16, 16)
                    a_ref[r, s] = a_ref[r, s] + b_ref[r, s]
                return 0

            lax.fori_loop(0, ch, row_body, 0)
            stores[b] = pltpu.async_copy(a_ref,
                                         out_hbm.at[pl.ds(base + c * ch, ch)],
                                         ssems[b])
            d0, d1 = nd0, nd1
        stores[0].wait()
        stores[1].wait()

    return k(ys, pos0, pos1)


def kernel(inputs, Wr, br, expert_embeddings, W1, b1, W2, b2, W3, b3):
    i0, i1, g0, g1 = _run_router(inputs, Wr, br, expert_embeddings)
    row_token, row_gate, blk_e, pos0, pos1 = _route_metadata(i0, i1, g0, g1)
    xs = _sc_gather(inputs, row_token)
    ys = _run_mlp(xs, row_gate, blk_e, W1, b1, W2, b2, W3, b3)
    return _sc_combine(ys, pos0, pos1)
